# baseline (speedup 1.0000x reference)
import functools

import jax
import jax.numpy as jnp
from jax import lax
from jax.experimental import pallas as pl
from jax.experimental.pallas import tpu as pltpu
from jax.experimental.pallas import tpu_sc as plsc

NC, NS = 2, 16
NW = NC * NS
CHUNK = 128
R = 2048
NGRAPHS = 64


def _round_up(v, m):
    return (v + m - 1) // m * m


def _make_scatter(n_rows_pad, d, units):
    rps = n_rows_pad // NS

    mesh = plsc.VectorSubcoreMesh(
        core_axis_name="c", subcore_axis_name="s", num_cores=NC, num_subcores=NS
    )

    @functools.partial(
        pl.kernel,
        out_type=jax.ShapeDtypeStruct((NC, n_rows_pad, d), jnp.float32),
        mesh=mesh,
        scratch_types=[
            pltpu.VMEM((units, CHUNK), jnp.int32),
            pltpu.VMEM((units, CHUNK), jnp.int32),
            pltpu.VMEM((CHUNK, d), jnp.float32),
            pltpu.VMEM_SHARED((n_rows_pad, d), jnp.float32),
            pltpu.SemaphoreType.DMA,
            pltpu.SemaphoreType.DMA,
        ],
    )
    def k(g_hbm, src2d, dst2d, zeros_hbm, out_hbm, sidx, didx, rows, acc_sh,
          gsem, ssem):
        c = lax.axis_index("c")
        s = lax.axis_index("s")
        rbase = (c * NS + s) * units

        pltpu.sync_copy(src2d.at[pl.ds(rbase, units)], sidx)
        pltpu.sync_copy(dst2d.at[pl.ds(rbase, units)], didx)
        pltpu.sync_copy(zeros_hbm, acc_sh.at[pl.ds(s * rps, rps)])
        plsc.subcore_barrier()

        def body(t, carry):
            g_cp = pltpu.make_async_copy(g_hbm.at[sidx.at[t]], rows, gsem)
            g_cp.start()
            g_cp.wait()
            s_cp = pltpu.make_async_copy(rows, acc_sh.at[didx.at[t]], ssem)
            s_cp.start(add=True)
            s_cp.wait()
            return carry

        lax.fori_loop(0, units, body, 0)
        plsc.subcore_barrier()
        pltpu.sync_copy(acc_sh.at[pl.ds(s * rps, rps)],
                        out_hbm.at[c, pl.ds(s * rps, rps)])

    return k


def _make_deg(n_rows_pad, d, units):
    assert units % 2 == 0
    rps = n_rows_pad // NS

    mesh = plsc.VectorSubcoreMesh(
        core_axis_name="c", subcore_axis_name="s", num_cores=NC, num_subcores=NS
    )

    @functools.partial(
        pl.kernel,
        out_type=jax.ShapeDtypeStruct((NC, n_rows_pad, d), jnp.float32),
        mesh=mesh,
        scratch_types=[
            pltpu.VMEM((units, CHUNK), jnp.int32),
            pltpu.VMEM((CHUNK, d), jnp.float32),
            pltpu.VMEM_SHARED((n_rows_pad, d), jnp.float32),
            pltpu.SemaphoreType.DMA,
            pltpu.SemaphoreType.DMA,
        ],
    )
    def k(dst2d, ones_hbm, zeros_hbm, out_hbm, didx, ones_v, acc_sh, s0, s1):
        c = lax.axis_index("c")
        s = lax.axis_index("s")
        rbase = (c * NS + s) * units

        pltpu.sync_copy(ones_hbm, ones_v)
        pltpu.sync_copy(dst2d.at[pl.ds(rbase, units)], didx)
        pltpu.sync_copy(zeros_hbm, acc_sh.at[pl.ds(s * rps, rps)])
        plsc.subcore_barrier()

        def pair(u, carry):
            t0 = 2 * u

            @pl.when(u > 0)
            def _():
                pltpu.make_async_copy(
                    ones_v, acc_sh.at[didx.at[t0 - 2]], s0).wait()
                pltpu.make_async_copy(
                    ones_v, acc_sh.at[didx.at[t0 - 1]], s1).wait()

            pltpu.make_async_copy(
                ones_v, acc_sh.at[didx.at[t0]], s0).start(add=True)
            pltpu.make_async_copy(
                ones_v, acc_sh.at[didx.at[t0 + 1]], s1).start(add=True)
            return carry

        lax.fori_loop(0, units // 2, pair, 0)
        pltpu.make_async_copy(ones_v, acc_sh.at[didx.at[units - 2]], s0).wait()
        pltpu.make_async_copy(ones_v, acc_sh.at[didx.at[units - 1]], s1).wait()
        plsc.subcore_barrier()
        pltpu.sync_copy(acc_sh.at[pl.ds(s * rps, rps)],
                        out_hbm.at[c, pl.ds(s * rps, rps)])

    return k


def _dinv_body(deg_ref, o_ref):
    o_ref[...] = lax.rsqrt(deg_ref[0] + deg_ref[1] + 1.0)


def _a1_body(dinv_ref, x_ref, w_ref, o_ref):
    o_ref[...] = dinv_ref[...] * jnp.dot(
        x_ref[...], w_ref[...], preferred_element_type=jnp.float32)


def _ab_body(dinv_ref, s_ref, g_ref, b_ref, w_ref, o_ref):
    dinv = dinv_ref[...]
    x = jnp.maximum(
        dinv * (s_ref[0] + s_ref[1] + g_ref[...]) + b_ref[...], 0.0)
    o_ref[...] = dinv * jnp.dot(x, w_ref[...],
                                preferred_element_type=jnp.float32)


def _c_body(dinv_ref, s_ref, g_ref, b_ref, batch_ref, wlin_ref, blin_ref,
            o_ref, acc, cnt):
    i = pl.program_id(0)

    @pl.when(i == 0)
    def _():
        acc[...] = jnp.zeros_like(acc)
        cnt[...] = jnp.zeros_like(cnt)

    dinv = dinv_ref[...]
    x = jnp.maximum(
        dinv * (s_ref[0] + s_ref[1] + g_ref[...]) + b_ref[...], 0.0)
    gid = lax.broadcasted_iota(jnp.int32, (R, NGRAPHS), 1)
    onehot = (batch_ref[...] == gid).astype(jnp.float32)
    dn = (((0,), (0,)), ((), ()))
    acc[...] += lax.dot_general(onehot, x, dn,
                                preferred_element_type=jnp.float32)
    cnt[...] += lax.dot_general(onehot, jnp.ones_like(x), dn,
                                preferred_element_type=jnp.float32)

    @pl.when(i == pl.num_programs(0) - 1)
    def _():
        mean = acc[...] / jnp.maximum(cnt[...], 1.0)
        o_ref[...] = jnp.dot(mean, wlin_ref[...],
                             preferred_element_type=jnp.float32) + blin_ref[...]


def _run_dinv(degp, n_rows_pad, d):
    grid = (n_rows_pad // R,)
    return pl.pallas_call(
        _dinv_body,
        grid=grid,
        in_specs=[pl.BlockSpec((NC, R, d), lambda i: (0, i, 0))],
        out_specs=pl.BlockSpec((R, d), lambda i: (i, 0)),
        out_shape=jax.ShapeDtypeStruct((n_rows_pad, d), jnp.float32),
    )(degp)


def _run_a1(dinv, x_pad, w, n_rows_pad, d):
    grid = (n_rows_pad // R,)
    return pl.pallas_call(
        _a1_body,
        grid=grid,
        in_specs=[
            pl.BlockSpec((R, d), lambda i: (i, 0)),
            pl.BlockSpec((R, d), lambda i: (i, 0)),
            pl.BlockSpec((d, d), lambda i: (0, 0)),
        ],
        out_specs=pl.BlockSpec((R, d), lambda i: (i, 0)),
        out_shape=jax.ShapeDtypeStruct((n_rows_pad, d), jnp.float32),
    )(dinv, x_pad, w)


def _run_ab(dinv, s, g, b, w, n_rows_pad, d):
    grid = (n_rows_pad // R,)
    return pl.pallas_call(
        _ab_body,
        grid=grid,
        in_specs=[
            pl.BlockSpec((R, d), lambda i: (i, 0)),
            pl.BlockSpec((NC, R, d), lambda i: (0, i, 0)),
            pl.BlockSpec((R, d), lambda i: (i, 0)),
            pl.BlockSpec((1, d), lambda i: (0, 0)),
            pl.BlockSpec((d, d), lambda i: (0, 0)),
        ],
        out_specs=pl.BlockSpec((R, d), lambda i: (i, 0)),
        out_shape=jax.ShapeDtypeStruct((n_rows_pad, d), jnp.float32),
    )(dinv, s, g, b, w)


def _run_c(dinv, s, g, b, batch64, wlin, blin, n_rows_pad, d, nout):
    grid = (n_rows_pad // R,)
    return pl.pallas_call(
        _c_body,
        grid=grid,
        in_specs=[
            pl.BlockSpec((R, d), lambda i: (i, 0)),
            pl.BlockSpec((NC, R, d), lambda i: (0, i, 0)),
            pl.BlockSpec((R, d), lambda i: (i, 0)),
            pl.BlockSpec((1, d), lambda i: (0, 0)),
            pl.BlockSpec((R, NGRAPHS), lambda i: (i, 0)),
            pl.BlockSpec((d, nout), lambda i: (0, 0)),
            pl.BlockSpec((1, nout), lambda i: (0, 0)),
        ],
        out_specs=pl.BlockSpec((NGRAPHS, nout), lambda i: (0, 0)),
        out_shape=jax.ShapeDtypeStruct((NGRAPHS, nout), jnp.float32),
        scratch_shapes=[
            pltpu.VMEM((NGRAPHS, d), jnp.float32),
            pltpu.VMEM((NGRAPHS, d), jnp.float32),
        ],
    )(dinv, s, g, b, batch64, wlin, blin)


def kernel(x, edge_index, batch, W1, b1, W2, b2, W3, b3, W4, b4, Wlin, blin):
    n, d = x.shape
    ne = edge_index.shape[1]
    nout = Wlin.shape[1]

    n_rows_pad = _round_up(n + 1, R)
    units = _round_up(-(-ne // (NW * CHUNK)), 2)
    ne_pad = NW * units * CHUNK
    rps = n_rows_pad // NS

    x_pad = jnp.pad(x.astype(jnp.float32), ((0, n_rows_pad - n), (0, 0)))
    ei = edge_index.astype(jnp.int32)
    pad_e = ne_pad - ne
    src = jnp.concatenate([ei[0], jnp.zeros((pad_e,), jnp.int32)])
    dst = jnp.concatenate([ei[1], jnp.full((pad_e,), n, jnp.int32)])
    src2d = src.reshape(-1, CHUNK)
    dst2d = dst.reshape(-1, CHUNK)
    zeros_d = jnp.zeros((rps, d), jnp.float32)
    ones_d = jnp.ones((CHUNK, d), jnp.float32)
    batch_pad = jnp.pad(batch.astype(jnp.int32), (0, n_rows_pad - n),
                        constant_values=NGRAPHS)
    batch64 = jnp.broadcast_to(batch_pad[:, None], (n_rows_pad, NGRAPHS))
    b1r, b2r, b3r, b4r = (v.reshape(1, d) for v in (b1, b2, b3, b4))
    blinr = blin.reshape(1, nout)

    deg_k = _make_deg(n_rows_pad, d, units)
    scat_k = _make_scatter(n_rows_pad, d, units)

    degp = deg_k(dst2d, ones_d, zeros_d)
    dinv = _run_dinv(degp, n_rows_pad, d)

    g = _run_a1(dinv, x_pad, W1, n_rows_pad, d)
    for w_next, b_prev in ((W2, b1r), (W3, b2r), (W4, b3r)):
        s = scat_k(g, src2d, dst2d, zeros_d)
        g = _run_ab(dinv, s, g, b_prev, w_next, n_rows_pad, d)
    s = scat_k(g, src2d, dst2d, zeros_d)
    return _run_c(dinv, s, g, b4r, batch64, Wlin, blinr, n_rows_pad, d, nout)


# speedup vs baseline: 1.1018x; 1.1018x over previous
import functools

import jax
import jax.numpy as jnp
from jax import lax
from jax.experimental import pallas as pl
from jax.experimental.pallas import tpu as pltpu
from jax.experimental.pallas import tpu_sc as plsc

NC, NS = 2, 16
NW = NC * NS
CHUNK = 128
R = 2048
NGRAPHS = 64


def _round_up(v, m):
    return (v + m - 1) // m * m


def _make_scatter(n_rows_pad, d, units):
    BCI = 8
    assert units % (2 * BCI) == 0
    nblk = units // BCI
    rps = n_rows_pad // NS

    mesh = plsc.VectorSubcoreMesh(
        core_axis_name="c", subcore_axis_name="s", num_cores=NC, num_subcores=NS
    )

    @functools.partial(
        pl.kernel,
        out_type=jax.ShapeDtypeStruct((NC, n_rows_pad, d), jnp.float32),
        mesh=mesh,
        scratch_types=[
            pltpu.VMEM((BCI, CHUNK), jnp.int32),
            pltpu.VMEM((BCI, CHUNK), jnp.int32),
            pltpu.VMEM((BCI, CHUNK), jnp.int32),
            pltpu.VMEM((BCI, CHUNK), jnp.int32),
            pltpu.VMEM((CHUNK, d), jnp.float32),
            pltpu.VMEM((CHUNK, d), jnp.float32),
            pltpu.VMEM_SHARED((n_rows_pad, d), jnp.float32),
            pltpu.SemaphoreType.DMA,
            pltpu.SemaphoreType.DMA,
            pltpu.SemaphoreType.DMA,
            pltpu.SemaphoreType.DMA,
            pltpu.SemaphoreType.DMA,
            pltpu.SemaphoreType.DMA,
        ],
    )
    def k(g_hbm, src2d, dst2d, zeros_hbm, out_hbm, sidx0, didx0, sidx1, didx1,
          rows0, rows1, acc_sh, g0, g1, s0, s1, i0, i1):
        c = lax.axis_index("c")
        s = lax.axis_index("s")
        rbase = (c * NS + s) * units

        pltpu.sync_copy(src2d.at[pl.ds(rbase, BCI)], sidx0)
        pltpu.sync_copy(dst2d.at[pl.ds(rbase, BCI)], didx0)
        pltpu.sync_copy(zeros_hbm, acc_sh.at[pl.ds(s * rps, rps)])
        plsc.subcore_barrier()

        def do_block(t, sic, dic, sio, dio):
            for v in range(BCI // 2):
                j0, j1 = 2 * v, 2 * v + 1
                if v > 0:
                    pltpu.make_async_copy(
                        rows0, acc_sh.at[dic.at[j0 - 2]], s0).wait()
                else:
                    @pl.when(t > 0)
                    def _():
                        pltpu.make_async_copy(
                            rows0, acc_sh.at[dio.at[BCI - 2]], s0).wait()
                pltpu.make_async_copy(g_hbm.at[sic.at[j0]], rows0, g0).start()
                if v > 0:
                    pltpu.make_async_copy(
                        rows1, acc_sh.at[dic.at[j1 - 2]], s1).wait()
                else:
                    @pl.when(t > 0)
                    def _():
                        pltpu.make_async_copy(
                            rows1, acc_sh.at[dio.at[BCI - 1]], s1).wait()
                pltpu.make_async_copy(g_hbm.at[sic.at[j1]], rows1, g1).start()
                if v == 1:
                    @pl.when(t + 1 < nblk)
                    def _():
                        rn = rbase + (t + 1) * BCI
                        pltpu.make_async_copy(
                            src2d.at[pl.ds(rn, BCI)], sio, i0).start()
                        pltpu.make_async_copy(
                            dst2d.at[pl.ds(rn, BCI)], dio, i1).start()
                pltpu.make_async_copy(g_hbm.at[sic.at[j0]], rows0, g0).wait()
                pltpu.make_async_copy(
                    rows0, acc_sh.at[dic.at[j0]], s0).start(add=True)
                pltpu.make_async_copy(g_hbm.at[sic.at[j1]], rows1, g1).wait()
                pltpu.make_async_copy(
                    rows1, acc_sh.at[dic.at[j1]], s1).start(add=True)

            @pl.when(t + 1 < nblk)
            def _():
                rn = rbase + (t + 1) * BCI
                pltpu.make_async_copy(src2d.at[pl.ds(rn, BCI)], sio, i0).wait()
                pltpu.make_async_copy(dst2d.at[pl.ds(rn, BCI)], dio, i1).wait()

        def blockpair(u, carry):
            do_block(2 * u, sidx0, didx0, sidx1, didx1)
            do_block(2 * u + 1, sidx1, didx1, sidx0, didx0)
            return carry

        lax.fori_loop(0, nblk // 2, blockpair, 0)
        pltpu.make_async_copy(rows0, acc_sh.at[didx1.at[BCI - 2]], s0).wait()
        pltpu.make_async_copy(rows1, acc_sh.at[didx1.at[BCI - 1]], s1).wait()
        plsc.subcore_barrier()
        pltpu.sync_copy(acc_sh.at[pl.ds(s * rps, rps)],
                        out_hbm.at[c, pl.ds(s * rps, rps)])

    return k


def _make_deg(n_rows_pad, d, units):
    assert units % 2 == 0
    rps = n_rows_pad // NS

    mesh = plsc.VectorSubcoreMesh(
        core_axis_name="c", subcore_axis_name="s", num_cores=NC, num_subcores=NS
    )

    @functools.partial(
        pl.kernel,
        out_type=jax.ShapeDtypeStruct((NC, n_rows_pad, d), jnp.float32),
        mesh=mesh,
        scratch_types=[
            pltpu.VMEM((units, CHUNK), jnp.int32),
            pltpu.VMEM((CHUNK, d), jnp.float32),
            pltpu.VMEM_SHARED((n_rows_pad, d), jnp.float32),
            pltpu.SemaphoreType.DMA,
            pltpu.SemaphoreType.DMA,
        ],
    )
    def k(dst2d, ones_hbm, zeros_hbm, out_hbm, didx, ones_v, acc_sh, s0, s1):
        c = lax.axis_index("c")
        s = lax.axis_index("s")
        rbase = (c * NS + s) * units

        pltpu.sync_copy(ones_hbm, ones_v)
        pltpu.sync_copy(dst2d.at[pl.ds(rbase, units)], didx)
        pltpu.sync_copy(zeros_hbm, acc_sh.at[pl.ds(s * rps, rps)])
        plsc.subcore_barrier()

        def pair(u, carry):
            t0 = 2 * u

            @pl.when(u > 0)
            def _():
                pltpu.make_async_copy(
                    ones_v, acc_sh.at[didx.at[t0 - 2]], s0).wait()
                pltpu.make_async_copy(
                    ones_v, acc_sh.at[didx.at[t0 - 1]], s1).wait()

            pltpu.make_async_copy(
                ones_v, acc_sh.at[didx.at[t0]], s0).start(add=True)
            pltpu.make_async_copy(
                ones_v, acc_sh.at[didx.at[t0 + 1]], s1).start(add=True)
            return carry

        lax.fori_loop(0, units // 2, pair, 0)
        pltpu.make_async_copy(ones_v, acc_sh.at[didx.at[units - 2]], s0).wait()
        pltpu.make_async_copy(ones_v, acc_sh.at[didx.at[units - 1]], s1).wait()
        plsc.subcore_barrier()
        pltpu.sync_copy(acc_sh.at[pl.ds(s * rps, rps)],
                        out_hbm.at[c, pl.ds(s * rps, rps)])

    return k


def _dinv_body(deg_ref, o_ref):
    o_ref[...] = lax.rsqrt(deg_ref[0] + deg_ref[1] + 1.0)


def _a1_body(dinv_ref, x_ref, w_ref, o_ref):
    o_ref[...] = dinv_ref[...] * jnp.dot(
        x_ref[...], w_ref[...], preferred_element_type=jnp.float32)


def _ab_body(dinv_ref, s_ref, g_ref, b_ref, w_ref, o_ref):
    dinv = dinv_ref[...]
    x = jnp.maximum(
        dinv * (s_ref[0] + s_ref[1] + g_ref[...]) + b_ref[...], 0.0)
    o_ref[...] = dinv * jnp.dot(x, w_ref[...],
                                preferred_element_type=jnp.float32)


def _c_body(dinv_ref, s_ref, g_ref, b_ref, batch_ref, wlin_ref, blin_ref,
            o_ref, acc, cnt):
    i = pl.program_id(0)

    @pl.when(i == 0)
    def _():
        acc[...] = jnp.zeros_like(acc)
        cnt[...] = jnp.zeros_like(cnt)

    dinv = dinv_ref[...]
    x = jnp.maximum(
        dinv * (s_ref[0] + s_ref[1] + g_ref[...]) + b_ref[...], 0.0)
    gid = lax.broadcasted_iota(jnp.int32, (R, NGRAPHS), 1)
    onehot = (batch_ref[...] == gid).astype(jnp.float32)
    dn = (((0,), (0,)), ((), ()))
    acc[...] += lax.dot_general(onehot, x, dn,
                                preferred_element_type=jnp.float32)
    cnt[...] += lax.dot_general(onehot, jnp.ones_like(x), dn,
                                preferred_element_type=jnp.float32)

    @pl.when(i == pl.num_programs(0) - 1)
    def _():
        mean = acc[...] / jnp.maximum(cnt[...], 1.0)
        o_ref[...] = jnp.dot(mean, wlin_ref[...],
                             preferred_element_type=jnp.float32) + blin_ref[...]


def _run_dinv(degp, n_rows_pad, d):
    grid = (n_rows_pad // R,)
    return pl.pallas_call(
        _dinv_body,
        grid=grid,
        in_specs=[pl.BlockSpec((NC, R, d), lambda i: (0, i, 0))],
        out_specs=pl.BlockSpec((R, d), lambda i: (i, 0)),
        out_shape=jax.ShapeDtypeStruct((n_rows_pad, d), jnp.float32),
    )(degp)


def _run_a1(dinv, x_pad, w, n_rows_pad, d):
    grid = (n_rows_pad // R,)
    return pl.pallas_call(
        _a1_body,
        grid=grid,
        in_specs=[
            pl.BlockSpec((R, d), lambda i: (i, 0)),
            pl.BlockSpec((R, d), lambda i: (i, 0)),
            pl.BlockSpec((d, d), lambda i: (0, 0)),
        ],
        out_specs=pl.BlockSpec((R, d), lambda i: (i, 0)),
        out_shape=jax.ShapeDtypeStruct((n_rows_pad, d), jnp.float32),
    )(dinv, x_pad, w)


def _run_ab(dinv, s, g, b, w, n_rows_pad, d):
    grid = (n_rows_pad // R,)
    return pl.pallas_call(
        _ab_body,
        grid=grid,
        in_specs=[
            pl.BlockSpec((R, d), lambda i: (i, 0)),
            pl.BlockSpec((NC, R, d), lambda i: (0, i, 0)),
            pl.BlockSpec((R, d), lambda i: (i, 0)),
            pl.BlockSpec((1, d), lambda i: (0, 0)),
            pl.BlockSpec((d, d), lambda i: (0, 0)),
        ],
        out_specs=pl.BlockSpec((R, d), lambda i: (i, 0)),
        out_shape=jax.ShapeDtypeStruct((n_rows_pad, d), jnp.float32),
    )(dinv, s, g, b, w)


def _run_c(dinv, s, g, b, batch64, wlin, blin, n_rows_pad, d, nout):
    grid = (n_rows_pad // R,)
    return pl.pallas_call(
        _c_body,
        grid=grid,
        in_specs=[
            pl.BlockSpec((R, d), lambda i: (i, 0)),
            pl.BlockSpec((NC, R, d), lambda i: (0, i, 0)),
            pl.BlockSpec((R, d), lambda i: (i, 0)),
            pl.BlockSpec((1, d), lambda i: (0, 0)),
            pl.BlockSpec((R, NGRAPHS), lambda i: (i, 0)),
            pl.BlockSpec((d, nout), lambda i: (0, 0)),
            pl.BlockSpec((1, nout), lambda i: (0, 0)),
        ],
        out_specs=pl.BlockSpec((NGRAPHS, nout), lambda i: (0, 0)),
        out_shape=jax.ShapeDtypeStruct((NGRAPHS, nout), jnp.float32),
        scratch_shapes=[
            pltpu.VMEM((NGRAPHS, d), jnp.float32),
            pltpu.VMEM((NGRAPHS, d), jnp.float32),
        ],
    )(dinv, s, g, b, batch64, wlin, blin)


def kernel(x, edge_index, batch, W1, b1, W2, b2, W3, b3, W4, b4, Wlin, blin):
    n, d = x.shape
    ne = edge_index.shape[1]
    nout = Wlin.shape[1]

    n_rows_pad = _round_up(n + 1, R)
    units = _round_up(-(-ne // (NW * CHUNK)), 16)
    ne_pad = NW * units * CHUNK
    rps = n_rows_pad // NS

    x_pad = jnp.pad(x.astype(jnp.float32), ((0, n_rows_pad - n), (0, 0)))
    ei = edge_index.astype(jnp.int32)
    pad_e = ne_pad - ne
    src = jnp.concatenate([ei[0], jnp.zeros((pad_e,), jnp.int32)])
    dst = jnp.concatenate([ei[1], jnp.full((pad_e,), n, jnp.int32)])
    src2d = src.reshape(-1, CHUNK)
    dst2d = dst.reshape(-1, CHUNK)
    zeros_d = jnp.zeros((rps, d), jnp.float32)
    ones_d = jnp.ones((CHUNK, d), jnp.float32)
    batch_pad = jnp.pad(batch.astype(jnp.int32), (0, n_rows_pad - n),
                        constant_values=NGRAPHS)
    batch64 = jnp.broadcast_to(batch_pad[:, None], (n_rows_pad, NGRAPHS))
    b1r, b2r, b3r, b4r = (v.reshape(1, d) for v in (b1, b2, b3, b4))
    blinr = blin.reshape(1, nout)

    deg_k = _make_deg(n_rows_pad, d, units)
    scat_k = _make_scatter(n_rows_pad, d, units)

    degp = deg_k(dst2d, ones_d, zeros_d)
    dinv = _run_dinv(degp, n_rows_pad, d)

    g = _run_a1(dinv, x_pad, W1, n_rows_pad, d)
    for w_next, b_prev in ((W2, b1r), (W3, b2r), (W4, b3r)):
        s = scat_k(g, src2d, dst2d, zeros_d)
        g = _run_ab(dinv, s, g, b_prev, w_next, n_rows_pad, d)
    s = scat_k(g, src2d, dst2d, zeros_d)
    return _run_c(dinv, s, g, b4r, batch64, Wlin, blinr, n_rows_pad, d, nout)


# speedup vs baseline: 2.7337x; 2.4811x over previous
import functools

import jax
import jax.numpy as jnp
from jax import lax
from jax.experimental import pallas as pl
from jax.experimental.pallas import tpu as pltpu
from jax.experimental.pallas import tpu_sc as plsc

NC, NS = 2, 16
NW = NC * NS
CHUNK = 125
R = 2048
NGRAPHS = 64


def _round_up(v, m):
    return (v + m - 1) // m * m


def _make_scatter(n_rows_pad, d, units):
    BCI = 8
    assert units % (2 * BCI) == 0
    nblk = units // BCI
    rps = n_rows_pad // NS

    mesh = plsc.VectorSubcoreMesh(
        core_axis_name="c", subcore_axis_name="s", num_cores=NC, num_subcores=NS
    )

    @functools.partial(
        pl.kernel,
        out_type=jax.ShapeDtypeStruct((NC, n_rows_pad, d), jnp.float32),
        mesh=mesh,
        scratch_types=[
            pltpu.VMEM((BCI, CHUNK), jnp.int32),
            pltpu.VMEM((BCI, CHUNK), jnp.int32),
            pltpu.VMEM((BCI, CHUNK), jnp.int32),
            pltpu.VMEM((BCI, CHUNK), jnp.int32),
            pltpu.VMEM((CHUNK, d), jnp.float32),
            pltpu.VMEM((CHUNK, d), jnp.float32),
            pltpu.VMEM_SHARED((n_rows_pad, d), jnp.float32),
            pltpu.SemaphoreType.DMA,
            pltpu.SemaphoreType.DMA,
            pltpu.SemaphoreType.DMA,
            pltpu.SemaphoreType.DMA,
            pltpu.SemaphoreType.DMA,
            pltpu.SemaphoreType.DMA,
        ],
    )
    def k(g_hbm, src2d, dst2d, zeros_hbm, out_hbm, sidx0, didx0, sidx1, didx1,
          rows0, rows1, acc_sh, g0, g1, s0, s1, i0, i1):
        c = lax.axis_index("c")
        s = lax.axis_index("s")
        rbase = (c * NS + s) * units

        pltpu.sync_copy(src2d.at[pl.ds(rbase, BCI)], sidx0)
        pltpu.sync_copy(dst2d.at[pl.ds(rbase, BCI)], didx0)
        pltpu.sync_copy(zeros_hbm, acc_sh.at[pl.ds(s * rps, rps)])
        plsc.subcore_barrier()

        def do_block(t, sic, dic, sio, dio):
            for v in range(BCI // 2):
                j0, j1 = 2 * v, 2 * v + 1
                if v > 0:
                    pltpu.make_async_copy(
                        rows0, acc_sh.at[dic.at[j0 - 2]], s0).wait()
                else:
                    @pl.when(t > 0)
                    def _():
                        pltpu.make_async_copy(
                            rows0, acc_sh.at[dio.at[BCI - 2]], s0).wait()
                pltpu.make_async_copy(g_hbm.at[sic.at[j0]], rows0, g0).start()
                if v > 0:
                    pltpu.make_async_copy(
                        rows1, acc_sh.at[dic.at[j1 - 2]], s1).wait()
                else:
                    @pl.when(t > 0)
                    def _():
                        pltpu.make_async_copy(
                            rows1, acc_sh.at[dio.at[BCI - 1]], s1).wait()
                pltpu.make_async_copy(g_hbm.at[sic.at[j1]], rows1, g1).start()
                if v == 1:
                    @pl.when(t + 1 < nblk)
                    def _():
                        rn = rbase + (t + 1) * BCI
                        pltpu.make_async_copy(
                            src2d.at[pl.ds(rn, BCI)], sio, i0).start()
                        pltpu.make_async_copy(
                            dst2d.at[pl.ds(rn, BCI)], dio, i1).start()
                pltpu.make_async_copy(g_hbm.at[sic.at[j0]], rows0, g0).wait()
                pltpu.make_async_copy(
                    rows0, acc_sh.at[dic.at[j0]], s0).start(add=True)
                pltpu.make_async_copy(g_hbm.at[sic.at[j1]], rows1, g1).wait()
                pltpu.make_async_copy(
                    rows1, acc_sh.at[dic.at[j1]], s1).start(add=True)

            @pl.when(t + 1 < nblk)
            def _():
                rn = rbase + (t + 1) * BCI
                pltpu.make_async_copy(src2d.at[pl.ds(rn, BCI)], sio, i0).wait()
                pltpu.make_async_copy(dst2d.at[pl.ds(rn, BCI)], dio, i1).wait()

        def blockpair(u, carry):
            do_block(2 * u, sidx0, didx0, sidx1, didx1)
            do_block(2 * u + 1, sidx1, didx1, sidx0, didx0)
            return carry

        lax.fori_loop(0, nblk // 2, blockpair, 0)
        pltpu.make_async_copy(rows0, acc_sh.at[didx1.at[BCI - 2]], s0).wait()
        pltpu.make_async_copy(rows1, acc_sh.at[didx1.at[BCI - 1]], s1).wait()
        plsc.subcore_barrier()
        pltpu.sync_copy(acc_sh.at[pl.ds(s * rps, rps)],
                        out_hbm.at[c, pl.ds(s * rps, rps)])

    return k


def _make_deg(n_rows_pad, d, units):
    assert units % 2 == 0
    rps = n_rows_pad // NS

    mesh = plsc.VectorSubcoreMesh(
        core_axis_name="c", subcore_axis_name="s", num_cores=NC, num_subcores=NS
    )

    @functools.partial(
        pl.kernel,
        out_type=jax.ShapeDtypeStruct((NC, n_rows_pad, d), jnp.float32),
        mesh=mesh,
        scratch_types=[
            pltpu.VMEM((units, CHUNK), jnp.int32),
            pltpu.VMEM((CHUNK, d), jnp.float32),
            pltpu.VMEM_SHARED((n_rows_pad, d), jnp.float32),
            pltpu.SemaphoreType.DMA,
            pltpu.SemaphoreType.DMA,
        ],
    )
    def k(dst2d, ones_hbm, zeros_hbm, out_hbm, didx, ones_v, acc_sh, s0, s1):
        c = lax.axis_index("c")
        s = lax.axis_index("s")
        rbase = (c * NS + s) * units

        pltpu.sync_copy(ones_hbm, ones_v)
        pltpu.sync_copy(dst2d.at[pl.ds(rbase, units)], didx)
        pltpu.sync_copy(zeros_hbm, acc_sh.at[pl.ds(s * rps, rps)])
        plsc.subcore_barrier()

        def pair(u, carry):
            t0 = 2 * u

            @pl.when(u > 0)
            def _():
                pltpu.make_async_copy(
                    ones_v, acc_sh.at[didx.at[t0 - 2]], s0).wait()
                pltpu.make_async_copy(
                    ones_v, acc_sh.at[didx.at[t0 - 1]], s1).wait()

            pltpu.make_async_copy(
                ones_v, acc_sh.at[didx.at[t0]], s0).start(add=True)
            pltpu.make_async_copy(
                ones_v, acc_sh.at[didx.at[t0 + 1]], s1).start(add=True)
            return carry

        lax.fori_loop(0, units // 2, pair, 0)
        pltpu.make_async_copy(ones_v, acc_sh.at[didx.at[units - 2]], s0).wait()
        pltpu.make_async_copy(ones_v, acc_sh.at[didx.at[units - 1]], s1).wait()
        plsc.subcore_barrier()
        pltpu.sync_copy(acc_sh.at[pl.ds(s * rps, rps)],
                        out_hbm.at[c, pl.ds(s * rps, rps)])

    return k


def _dinv_body(deg_ref, o_ref):
    o_ref[...] = lax.rsqrt(deg_ref[0] + deg_ref[1] + 1.0)


def _a1_body(dinv_ref, x_ref, w_ref, o_ref):
    o_ref[...] = dinv_ref[...] * jnp.dot(
        x_ref[...], w_ref[...], preferred_element_type=jnp.float32)


def _ab_body(dinv_ref, s_ref, g_ref, b_ref, w_ref, o_ref):
    dinv = dinv_ref[...]
    x = jnp.maximum(
        dinv * (s_ref[0] + s_ref[1] + g_ref[...]) + b_ref[...], 0.0)
    o_ref[...] = dinv * jnp.dot(x, w_ref[...],
                                preferred_element_type=jnp.float32)


def _c_body(dinv_ref, s_ref, g_ref, b_ref, batch_ref, wlin_ref, blin_ref,
            o_ref, acc, cnt):
    i = pl.program_id(0)

    @pl.when(i == 0)
    def _():
        acc[...] = jnp.zeros_like(acc)
        cnt[...] = jnp.zeros_like(cnt)

    dinv = dinv_ref[...]
    x = jnp.maximum(
        dinv * (s_ref[0] + s_ref[1] + g_ref[...]) + b_ref[...], 0.0)
    gid = lax.broadcasted_iota(jnp.int32, (R, NGRAPHS), 1)
    onehot = (batch_ref[...] == gid).astype(jnp.float32)
    dn = (((0,), (0,)), ((), ()))
    acc[...] += lax.dot_general(onehot, x, dn,
                                preferred_element_type=jnp.float32)
    cnt[...] += lax.dot_general(onehot, jnp.ones_like(x), dn,
                                preferred_element_type=jnp.float32)

    @pl.when(i == pl.num_programs(0) - 1)
    def _():
        mean = acc[...] / jnp.maximum(cnt[...], 1.0)
        o_ref[...] = jnp.dot(mean, wlin_ref[...],
                             preferred_element_type=jnp.float32) + blin_ref[...]


def _run_dinv(degp, n_rows_pad, d):
    grid = (n_rows_pad // R,)
    return pl.pallas_call(
        _dinv_body,
        grid=grid,
        in_specs=[pl.BlockSpec((NC, R, d), lambda i: (0, i, 0))],
        out_specs=pl.BlockSpec((R, d), lambda i: (i, 0)),
        out_shape=jax.ShapeDtypeStruct((n_rows_pad, d), jnp.float32),
    )(degp)


def _run_a1(dinv, x_pad, w, n_rows_pad, d):
    grid = (n_rows_pad // R,)
    return pl.pallas_call(
        _a1_body,
        grid=grid,
        in_specs=[
            pl.BlockSpec((R, d), lambda i: (i, 0)),
            pl.BlockSpec((R, d), lambda i: (i, 0)),
            pl.BlockSpec((d, d), lambda i: (0, 0)),
        ],
        out_specs=pl.BlockSpec((R, d), lambda i: (i, 0)),
        out_shape=jax.ShapeDtypeStruct((n_rows_pad, d), jnp.float32),
    )(dinv, x_pad, w)


def _run_ab(dinv, s, g, b, w, n_rows_pad, d):
    grid = (n_rows_pad // R,)
    return pl.pallas_call(
        _ab_body,
        grid=grid,
        in_specs=[
            pl.BlockSpec((R, d), lambda i: (i, 0)),
            pl.BlockSpec((NC, R, d), lambda i: (0, i, 0)),
            pl.BlockSpec((R, d), lambda i: (i, 0)),
            pl.BlockSpec((1, d), lambda i: (0, 0)),
            pl.BlockSpec((d, d), lambda i: (0, 0)),
        ],
        out_specs=pl.BlockSpec((R, d), lambda i: (i, 0)),
        out_shape=jax.ShapeDtypeStruct((n_rows_pad, d), jnp.float32),
    )(dinv, s, g, b, w)


def _run_c(dinv, s, g, b, batch64, wlin, blin, n_rows_pad, d, nout):
    grid = (n_rows_pad // R,)
    return pl.pallas_call(
        _c_body,
        grid=grid,
        in_specs=[
            pl.BlockSpec((R, d), lambda i: (i, 0)),
            pl.BlockSpec((NC, R, d), lambda i: (0, i, 0)),
            pl.BlockSpec((R, d), lambda i: (i, 0)),
            pl.BlockSpec((1, d), lambda i: (0, 0)),
            pl.BlockSpec((R, NGRAPHS), lambda i: (i, 0)),
            pl.BlockSpec((d, nout), lambda i: (0, 0)),
            pl.BlockSpec((1, nout), lambda i: (0, 0)),
        ],
        out_specs=pl.BlockSpec((NGRAPHS, nout), lambda i: (0, 0)),
        out_shape=jax.ShapeDtypeStruct((NGRAPHS, nout), jnp.float32),
        scratch_shapes=[
            pltpu.VMEM((NGRAPHS, d), jnp.float32),
            pltpu.VMEM((NGRAPHS, d), jnp.float32),
        ],
    )(dinv, s, g, b, batch64, wlin, blin)


def kernel(x, edge_index, batch, W1, b1, W2, b2, W3, b3, W4, b4, Wlin, blin):
    n, d = x.shape
    ne = edge_index.shape[1]
    nout = Wlin.shape[1]

    n_rows_pad = _round_up(n + 1, R)
    units = _round_up(-(-ne // (NW * CHUNK)), 16)
    ne_pad = NW * units * CHUNK
    rps = n_rows_pad // NS

    x_pad = jnp.pad(x.astype(jnp.float32), ((0, n_rows_pad - n), (0, 0)))
    ei = edge_index.astype(jnp.int32)
    pad_e = ne_pad - ne
    src = jnp.concatenate([ei[0], jnp.zeros((pad_e,), jnp.int32)])
    dst = jnp.concatenate([ei[1], jnp.full((pad_e,), n, jnp.int32)])
    src2d = src.reshape(-1, CHUNK)
    dst2d = dst.reshape(-1, CHUNK)
    zeros_d = jnp.zeros((rps, d), jnp.float32)
    ones_d = jnp.ones((CHUNK, d), jnp.float32)
    batch_pad = jnp.pad(batch.astype(jnp.int32), (0, n_rows_pad - n),
                        constant_values=NGRAPHS)
    batch64 = jnp.broadcast_to(batch_pad[:, None], (n_rows_pad, NGRAPHS))
    b1r, b2r, b3r, b4r = (v.reshape(1, d) for v in (b1, b2, b3, b4))
    blinr = blin.reshape(1, nout)

    deg_k = _make_deg(n_rows_pad, d, units)
    scat_k = _make_scatter(n_rows_pad, d, units)

    degp = deg_k(dst2d, ones_d, zeros_d)
    dinv = _run_dinv(degp, n_rows_pad, d)

    g = _run_a1(dinv, x_pad, W1, n_rows_pad, d)
    for w_next, b_prev in ((W2, b1r), (W3, b2r), (W4, b3r)):
        s = scat_k(g, src2d, dst2d, zeros_d)
        g = _run_ab(dinv, s, g, b_prev, w_next, n_rows_pad, d)
    s = scat_k(g, src2d, dst2d, zeros_d)
    return _run_c(dinv, s, g, b4r, batch64, Wlin, blinr, n_rows_pad, d, nout)
